# baseline (speedup 1.0000x reference)
import jax
import jax.numpy as jnp
from jax.experimental import pallas as pl
from jax.experimental.pallas import tpu as pltpu

EMB = 128
KNN = 4
TAU = 0.3
BIG = 3.0e38


def _erf(x):
    a1, a2, a3, a4, a5 = (0.254829592, -0.284496736, 1.421413741,
                          -1.453152027, 1.061405429)
    p = 0.3275911
    ax = jnp.abs(x)
    t = 1.0 / (1.0 + p * ax)
    poly = t * (a1 + t * (a2 + t * (a3 + t * (a4 + t * a5))))
    y = 1.0 - poly * jnp.exp(-ax * ax)
    return jnp.sign(x) * y


def _gelu(x):
    return 0.5 * x * (1.0 + _erf(x * 0.7071067811865476))


def _body(ax_ref, ay_ref, gl_ref, w1t_ref, b1_ref, w2t_ref, b2_ref,
          out_ref):
    bq = ax_ref.shape[0]
    m = ax_ref.shape[1]
    nchunks = m // 128

    gl = gl_ref[...]
    qx = gl[:, 0:1]
    qy = gl[:, 1:2]
    lane = jax.lax.broadcasted_iota(jnp.int32, (bq, 128), 1)

    ms = [jnp.full((bq, 128), BIG, jnp.float32) for _ in range(KNN)]
    xs = [jnp.zeros((bq, 128), jnp.float32) for _ in range(KNN)]
    ys = [jnp.zeros((bq, 128), jnp.float32) for _ in range(KNN)]

    for c in range(nchunks):
        axc = ax_ref[:, c * 128:(c + 1) * 128]
        ayc = ay_ref[:, c * 128:(c + 1) * 128]
        dx = axc - qx
        dy = ayc - qy
        key = dx * dx + dy * dy
        px = axc
        py = ayc
        for i in range(KNN):
            cond = key < ms[i]
            nm = jnp.where(cond, key, ms[i])
            nx = jnp.where(cond, px, xs[i])
            ny = jnp.where(cond, py, ys[i])
            if i < KNN - 1:
                key = jnp.where(cond, ms[i], key)
                px = jnp.where(cond, xs[i], px)
                py = jnp.where(cond, ys[i], py)
            ms[i], xs[i], ys[i] = nm, nx, ny

    vals, tx, ty = [], [], []
    for k in range(KNN):
        rmin = jnp.min(ms[0], axis=1, keepdims=True)
        hit = ms[0] == rmin
        first = jnp.min(jnp.where(hit, lane, 128), axis=1, keepdims=True)
        h1 = lane == first
        tx.append(jnp.sum(jnp.where(h1, xs[0], 0.0), axis=1, keepdims=True))
        ty.append(jnp.sum(jnp.where(h1, ys[0], 0.0), axis=1, keepdims=True))
        vals.append(rmin)
        if k < KNN - 1:
            for i in range(KNN - 1):
                ms[i] = jnp.where(h1, ms[i + 1], ms[i])
                xs[i] = jnp.where(h1, xs[i + 1], xs[i])
                ys[i] = jnp.where(h1, ys[i + 1], ys[i])

    v = jnp.concatenate(vals, axis=1)
    logits = v * (1.0 / TAU)
    mx = jnp.max(logits, axis=1, keepdims=True)
    e = jnp.exp(logits - mx)
    inv_se = 1.0 / jnp.sum(e, axis=1, keepdims=True)

    w1t = w1t_ref[...]
    w1x = w1t[0:1, :]
    w1y = w1t[1:2, :]
    b1 = b1_ref[...]
    w2t = w2t_ref[...]
    b2 = b2_ref[...]

    acc = jnp.zeros((bq, EMB), jnp.float32)
    for k in range(KNN):
        h1v = _gelu(tx[k] * w1x + ty[k] * w1y + b1)
        h2 = jnp.dot(h1v, w2t, preferred_element_type=jnp.float32) + b2
        h2 = _gelu(h2)
        acc = acc + h2 * (e[:, k:k + 1] * inv_se)
    out_ref[...] = acc


def _call(ax, ay, gl, w1t, b1r, w2t, b2r):
    B, M = ax.shape
    bq = min(512, B)
    grid = (B // bq,)
    return pl.pallas_call(
        _body,
        grid=grid,
        in_specs=[
            pl.BlockSpec((bq, M), lambda i: (i, 0)),
            pl.BlockSpec((bq, M), lambda i: (i, 0)),
            pl.BlockSpec((bq, 2), lambda i: (i, 0)),
            pl.BlockSpec((2, EMB), lambda i: (0, 0)),
            pl.BlockSpec((1, EMB), lambda i: (0, 0)),
            pl.BlockSpec((EMB, EMB), lambda i: (0, 0)),
            pl.BlockSpec((1, EMB), lambda i: (0, 0)),
        ],
        out_specs=pl.BlockSpec((bq, EMB), lambda i: (i, 0)),
        out_shape=jax.ShapeDtypeStruct((B, EMB), jnp.float32),
    )(ax, ay, gl, w1t, b1r, w2t, b2r)


@jax.jit
def kernel(Gl_cur, ancL, W1, b1, W2, b2):
    B, M, _ = ancL.shape
    w1t = W1.T
    w2t = W2.T
    b1r = b1.reshape(1, EMB)
    b2r = b2.reshape(1, EMB)

    S = 4 if B % 4 == 0 else 1
    sb = B // S
    outs = []
    for s in range(S):
        anc_s = jax.lax.slice_in_dim(ancL, s * sb, (s + 1) * sb, axis=0)
        gl_s = jax.lax.slice_in_dim(Gl_cur, s * sb, (s + 1) * sb, axis=0)
        outs.append(_call(anc_s[:, :, 0], anc_s[:, :, 1], gl_s,
                          w1t, b1r, w2t, b2r))
    return jnp.concatenate(outs, axis=0)


# speedup vs baseline: 1.0617x; 1.0617x over previous
import jax
import jax.numpy as jnp
from jax.experimental import pallas as pl
from jax.experimental.pallas import tpu as pltpu

EMB = 128
KNN = 4
TAU = 0.3
BIG = 3.0e38


def _erf(x):
    a1, a2, a3, a4, a5 = (0.254829592, -0.284496736, 1.421413741,
                          -1.453152027, 1.061405429)
    p = 0.3275911
    ax = jnp.abs(x)
    t = 1.0 / (1.0 + p * ax)
    poly = t * (a1 + t * (a2 + t * (a3 + t * (a4 + t * a5))))
    y = 1.0 - poly * jnp.exp(-ax * ax)
    return jnp.sign(x) * y


def _gelu(x):
    return 0.5 * x * (1.0 + _erf(x * 0.7071067811865476))


def _body(ax_ref, ay_ref, gl_ref, w1t_ref, b1_ref, w2t_ref, b2_ref,
          out_ref):
    bq = ax_ref.shape[0]
    m = ax_ref.shape[1]
    nchunks = m // 128

    gl = gl_ref[...]
    qx = gl[:, 0:1]
    qy = gl[:, 1:2]
    lane = jax.lax.broadcasted_iota(jnp.int32, (bq, 128), 1)

    ms = [jnp.full((bq, 128), BIG, jnp.float32) for _ in range(KNN)]
    xs = [jnp.zeros((bq, 128), jnp.float32) for _ in range(KNN)]
    ys = [jnp.zeros((bq, 128), jnp.float32) for _ in range(KNN)]

    for c in range(nchunks):
        axc = ax_ref[:, c * 128:(c + 1) * 128]
        ayc = ay_ref[:, c * 128:(c + 1) * 128]
        dx = axc - qx
        dy = ayc - qy
        key = dx * dx + dy * dy
        px = axc
        py = ayc
        for i in range(KNN):
            cond = key < ms[i]
            nm = jnp.where(cond, key, ms[i])
            nx = jnp.where(cond, px, xs[i])
            ny = jnp.where(cond, py, ys[i])
            if i < KNN - 1:
                key = jnp.where(cond, ms[i], key)
                px = jnp.where(cond, xs[i], px)
                py = jnp.where(cond, ys[i], py)
            ms[i], xs[i], ys[i] = nm, nx, ny

    vals, tx, ty = [], [], []
    for k in range(KNN):
        rmin = jnp.min(ms[0], axis=1, keepdims=True)
        hit = ms[0] == rmin
        first = jnp.min(jnp.where(hit, lane, 128), axis=1, keepdims=True)
        h1 = lane == first
        tx.append(jnp.sum(jnp.where(h1, xs[0], 0.0), axis=1, keepdims=True))
        ty.append(jnp.sum(jnp.where(h1, ys[0], 0.0), axis=1, keepdims=True))
        vals.append(rmin)
        if k < KNN - 1:
            for i in range(KNN - 1):
                ms[i] = jnp.where(h1, ms[i + 1], ms[i])
                xs[i] = jnp.where(h1, xs[i + 1], xs[i])
                ys[i] = jnp.where(h1, ys[i + 1], ys[i])

    v = jnp.concatenate(vals, axis=1)
    logits = v * (1.0 / TAU)
    mx = jnp.max(logits, axis=1, keepdims=True)
    e = jnp.exp(logits - mx)
    inv_se = 1.0 / jnp.sum(e, axis=1, keepdims=True)

    w1t = w1t_ref[...]
    w1x = w1t[0:1, :]
    w1y = w1t[1:2, :]
    b1 = b1_ref[...]
    w2t = w2t_ref[...]
    b2 = b2_ref[...]

    acc = jnp.zeros((bq, EMB), jnp.float32)
    for k in range(KNN):
        h1v = _gelu(tx[k] * w1x + ty[k] * w1y + b1)
        h2 = jnp.dot(h1v, w2t, preferred_element_type=jnp.float32) + b2
        h2 = _gelu(h2)
        acc = acc + h2 * (e[:, k:k + 1] * inv_se)
    out_ref[...] = acc


@jax.jit
def kernel(Gl_cur, ancL, W1, b1, W2, b2):
    B, M, _ = ancL.shape
    ax = ancL[:, :, 0]
    ay = ancL[:, :, 1]
    w1t = W1.T
    w2t = W2.T
    b1r = b1.reshape(1, EMB)
    b2r = b2.reshape(1, EMB)

    bq = min(1024, B)
    grid = (B // bq,)
    return pl.pallas_call(
        _body,
        grid=grid,
        compiler_params=pltpu.CompilerParams(
            vmem_limit_bytes=100 * 1024 * 1024),
        in_specs=[
            pl.BlockSpec((bq, M), lambda i: (i, 0)),
            pl.BlockSpec((bq, M), lambda i: (i, 0)),
            pl.BlockSpec((bq, 2), lambda i: (i, 0)),
            pl.BlockSpec((2, EMB), lambda i: (0, 0)),
            pl.BlockSpec((1, EMB), lambda i: (0, 0)),
            pl.BlockSpec((EMB, EMB), lambda i: (0, 0)),
            pl.BlockSpec((1, EMB), lambda i: (0, 0)),
        ],
        out_specs=pl.BlockSpec((bq, EMB), lambda i: (i, 0)),
        out_shape=jax.ShapeDtypeStruct((B, EMB), jnp.float32),
    )(ax, ay, Gl_cur, w1t, b1r, w2t, b2r)


# speedup vs baseline: 1.4210x; 1.3384x over previous
import jax
import jax.numpy as jnp
from jax.experimental import pallas as pl
from jax.experimental.pallas import tpu as pltpu

EMB = 128
KNN = 4
TAU = 0.3
BIG = 3.0e38


def _erf(x):
    a1, a2, a3, a4, a5 = (0.254829592, -0.284496736, 1.421413741,
                          -1.453152027, 1.061405429)
    p = 0.3275911
    ax = jnp.abs(x)
    t = 1.0 / (1.0 + p * ax)
    poly = t * (a1 + t * (a2 + t * (a3 + t * (a4 + t * a5))))
    y = 1.0 - poly * jnp.exp(-ax * ax)
    return jnp.sign(x) * y


def _gelu(x):
    return 0.5 * x * (1.0 + _erf(x * 0.7071067811865476))


def _body(a_ref, gl_ref, w1t_ref, b1_ref, w2t_ref, b2_ref,
          out_ref):
    bq = a_ref.shape[1]
    m = a_ref.shape[2]
    nchunks = m // 128

    gl = gl_ref[...]
    qx = gl[:, 0:1]
    qy = gl[:, 1:2]
    lane = jax.lax.broadcasted_iota(jnp.int32, (bq, 128), 1)

    ms = [jnp.full((bq, 128), BIG, jnp.float32) for _ in range(KNN)]
    xs = [jnp.zeros((bq, 128), jnp.float32) for _ in range(KNN)]
    ys = [jnp.zeros((bq, 128), jnp.float32) for _ in range(KNN)]

    for c in range(nchunks):
        axc = a_ref[0, :, c * 128:(c + 1) * 128]
        ayc = a_ref[1, :, c * 128:(c + 1) * 128]
        dx = axc - qx
        dy = ayc - qy
        key = dx * dx + dy * dy
        px = axc
        py = ayc
        for i in range(KNN):
            cond = key < ms[i]
            nm = jnp.where(cond, key, ms[i])
            nx = jnp.where(cond, px, xs[i])
            ny = jnp.where(cond, py, ys[i])
            if i < KNN - 1:
                key = jnp.where(cond, ms[i], key)
                px = jnp.where(cond, xs[i], px)
                py = jnp.where(cond, ys[i], py)
            ms[i], xs[i], ys[i] = nm, nx, ny

    vals, tx, ty = [], [], []
    for k in range(KNN):
        rmin = jnp.min(ms[0], axis=1, keepdims=True)
        hit = ms[0] == rmin
        first = jnp.min(jnp.where(hit, lane, 128), axis=1, keepdims=True)
        h1 = lane == first
        tx.append(jnp.sum(jnp.where(h1, xs[0], 0.0), axis=1, keepdims=True))
        ty.append(jnp.sum(jnp.where(h1, ys[0], 0.0), axis=1, keepdims=True))
        vals.append(rmin)
        if k < KNN - 1:
            for i in range(KNN - 1):
                ms[i] = jnp.where(h1, ms[i + 1], ms[i])
                xs[i] = jnp.where(h1, xs[i + 1], xs[i])
                ys[i] = jnp.where(h1, ys[i + 1], ys[i])

    v = jnp.concatenate(vals, axis=1)
    logits = v * (1.0 / TAU)
    mx = jnp.max(logits, axis=1, keepdims=True)
    e = jnp.exp(logits - mx)
    inv_se = 1.0 / jnp.sum(e, axis=1, keepdims=True)

    w1t = w1t_ref[...]
    w1x = w1t[0:1, :]
    w1y = w1t[1:2, :]
    b1 = b1_ref[...]
    w2t = w2t_ref[...]
    b2 = b2_ref[...]

    acc = jnp.zeros((bq, EMB), jnp.float32)
    for k in range(KNN):
        h1v = _gelu(tx[k] * w1x + ty[k] * w1y + b1)
        h2 = jnp.dot(h1v, w2t, preferred_element_type=jnp.float32) + b2
        h2 = _gelu(h2)
        acc = acc + h2 * (e[:, k:k + 1] * inv_se)
    out_ref[...] = acc


@jax.jit
def kernel(Gl_cur, ancL, W1, b1, W2, b2):
    B, M, _ = ancL.shape
    axy = jnp.transpose(ancL, (2, 0, 1))
    w1t = W1.T
    w2t = W2.T
    b1r = b1.reshape(1, EMB)
    b2r = b2.reshape(1, EMB)

    bq = min(1024, B)
    grid = (B // bq,)
    return pl.pallas_call(
        _body,
        grid=grid,
        compiler_params=pltpu.CompilerParams(
            vmem_limit_bytes=100 * 1024 * 1024),
        in_specs=[
            pl.BlockSpec((2, bq, M), lambda i: (0, i, 0)),
            pl.BlockSpec((bq, 2), lambda i: (i, 0)),
            pl.BlockSpec((2, EMB), lambda i: (0, 0)),
            pl.BlockSpec((1, EMB), lambda i: (0, 0)),
            pl.BlockSpec((EMB, EMB), lambda i: (0, 0)),
            pl.BlockSpec((1, EMB), lambda i: (0, 0)),
        ],
        out_specs=pl.BlockSpec((bq, EMB), lambda i: (i, 0)),
        out_shape=jax.ShapeDtypeStruct((B, EMB), jnp.float32),
    )(axy, Gl_cur, w1t, b1r, w2t, b2r)
